# baseline (speedup 1.0000x reference)
import functools

import jax
import jax.numpy as jnp
from jax import lax
from jax.experimental import pallas as pl
from jax.experimental.pallas import tpu as pltpu
from jax.experimental.pallas import tpu_sc as plsc

NC = 2
NS = 16
NW = NC * NS
LANES = 128
G = 64


def _round_up(v, m):
    return -(-v // m) * m


def _sc_degree(dst3, ones_blk, zeros_n8, n):
    CH = dst3.shape[1]
    n_pad = _round_up(n, NS * 8)
    ZR = n_pad // NS
    mesh = plsc.VectorSubcoreMesh(
        core_axis_name="c", subcore_axis_name="s", num_cores=NC, num_subcores=NS)

    @functools.partial(
        pl.kernel,
        out_type=jax.ShapeDtypeStruct((NC, n_pad, 8), jnp.float32),
        mesh=mesh,
        compiler_params=pltpu.CompilerParams(use_tc_tiling_on_sc=False),
        scratch_types=[
            pltpu.VMEM((CH, LANES), jnp.int32),
            pltpu.VMEM((LANES, 8), jnp.float32),
            pltpu.VMEM_SHARED((n_pad, 8), jnp.float32),
        ],
    )
    def deg_kernel(dst_hbm, ones_hbm, zeros_hbm, out_hbm, dst_v, ones_v, deg_sh):
        cid = lax.axis_index("c")
        sid = lax.axis_index("s")
        wid = sid * NC + cid
        pltpu.sync_copy(zeros_hbm.at[pl.ds(sid * ZR, ZR)],
                        deg_sh.at[pl.ds(sid * ZR, ZR)])
        pltpu.sync_copy(ones_hbm, ones_v)
        pltpu.sync_copy(dst_hbm.at[wid], dst_v)
        plsc.subcore_barrier()

        @pl.loop(0, CH)
        def _(j):
            pltpu.sync_copy(ones_v, deg_sh.at[dst_v.at[j]], add=True)

        plsc.subcore_barrier()
        pltpu.sync_copy(deg_sh.at[pl.ds(sid * ZR, ZR)],
                        out_hbm.at[cid, pl.ds(sid * ZR, ZR)])

    return deg_kernel(dst3, ones_blk, zeros_n8)


def _sc_scatter_rows(hs_split, src3, dst3, zeros_nhh, n, h):
    CB, KL = dst3.shape[1], dst3.shape[2]
    hh = h // NC
    n_pad = _round_up(n, NS * 8)
    ZR = n_pad // NS
    mesh = plsc.VectorSubcoreMesh(
        core_axis_name="c", subcore_axis_name="s", num_cores=NC, num_subcores=NS)

    @functools.partial(
        pl.kernel,
        out_type=jax.ShapeDtypeStruct((NC, n_pad, hh), jnp.float32),
        mesh=mesh,
        compiler_params=pltpu.CompilerParams(use_tc_tiling_on_sc=False),
        scratch_types=[
            pltpu.VMEM((CB + 1, KL), jnp.int32),
            pltpu.VMEM((CB, KL), jnp.int32),
            [pltpu.VMEM((KL, hh), jnp.float32) for _ in range(2)],
            pltpu.VMEM_SHARED((n_pad, hh), jnp.float32),
            pltpu.VMEM_SHARED((n_pad, hh), jnp.float32),
            [pltpu.SemaphoreType.DMA for _ in range(2)],
            [pltpu.SemaphoreType.DMA for _ in range(2)],
        ],
    )
    def msg_kernel(hs_hbm, src_hbm, dst_hbm, zeros_hbm, out_hbm,
                   src_v, dst_v, rows, hs_sh, acc_sh, semg, sems):
        cid = lax.axis_index("c")
        sid = lax.axis_index("s")
        pltpu.sync_copy(hs_hbm.at[cid, pl.ds(sid * ZR, ZR)],
                        hs_sh.at[pl.ds(sid * ZR, ZR)])
        pltpu.sync_copy(zeros_hbm.at[pl.ds(sid * ZR, ZR)],
                        acc_sh.at[pl.ds(sid * ZR, ZR)])
        pltpu.sync_copy(src_hbm.at[sid], src_v)
        pltpu.sync_copy(dst_hbm.at[sid], dst_v)
        plsc.subcore_barrier()

        def gath(c, b):
            pltpu.async_copy(hs_sh.at[src_v.at[c]], rows[b], semg[b])

        def gath_wait(c, b):
            pltpu.make_async_copy(hs_sh.at[src_v.at[c]], rows[b],
                                  semg[b]).wait()

        def scat(c, b):
            pltpu.async_copy(rows[b], acc_sh.at[dst_v.at[c]], sems[b],
                             add=True)

        def scat_wait(c, b):
            pltpu.make_async_copy(rows[b], acc_sh.at[dst_v.at[c]],
                                  sems[b]).wait()

        gath(0, 0)

        @pl.loop(0, CB, step=2)
        def _(j):
            @pl.when(j > 0)
            def _():
                scat_wait(j - 1, 1)

            gath(j + 1, 1)
            gath_wait(j, 0)
            scat(j, 0)
            scat_wait(j, 0)
            gath(j + 2, 0)
            gath_wait(j + 1, 1)
            scat(j + 1, 1)

        gath_wait(CB, 0)
        scat_wait(CB - 1, 1)
        plsc.subcore_barrier()
        pltpu.sync_copy(acc_sh.at[pl.ds(sid * ZR, ZR)],
                        out_hbm.at[cid, pl.ds(sid * ZR, ZR)])

    return msg_kernel(hs_split, src3, dst3, zeros_nhh)


def _split_cols(a, n_pad):
    return a.reshape(n_pad, NC, a.shape[1] // NC).transpose(1, 0, 2)


def _tc_matmul_scale(x, W, degp):
    n, d = x.shape
    h = W.shape[1]
    R = 2000

    def body(x_ref, w_ref, deg_ref, hs_ref, dinv_ref):
        deg = deg_ref[0, :, 0:1] + deg_ref[1, :, 0:1] + 1.0
        dinv = lax.rsqrt(deg)
        hm = jnp.dot(x_ref[...], w_ref[...], preferred_element_type=jnp.float32)
        hs_ref[...] = hm * dinv
        dinv_ref[...] = dinv

    return pl.pallas_call(
        body,
        grid=(n // R,),
        in_specs=[
            pl.BlockSpec((R, d), lambda i: (i, 0)),
            pl.BlockSpec((d, h), lambda i: (0, 0)),
            pl.BlockSpec((NC, R, 8), lambda i: (0, i, 0)),
        ],
        out_specs=[
            pl.BlockSpec((R, h), lambda i: (i, 0)),
            pl.BlockSpec((R, 1), lambda i: (i, 0)),
        ],
        out_shape=[
            jax.ShapeDtypeStruct((n, h), jnp.float32),
            jax.ShapeDtypeStruct((n, 1), jnp.float32),
        ],
    )(x, W, degp)


def _tc_layer(acc, hs, dinv, b, W):
    n, h = hs.shape
    hh = h // NC
    h2 = W.shape[1]
    R = 2000

    def body(acc_ref, hs_ref, dinv_ref, b_ref, w_ref, out_ref):
        s = jnp.concatenate([acc_ref[0], acc_ref[1]], axis=1) + hs_ref[...]
        h1 = jnp.maximum(s * dinv_ref[...] + b_ref[...], 0.0)
        out_ref[...] = jnp.dot(h1, w_ref[...],
                               preferred_element_type=jnp.float32) * dinv_ref[...]

    return pl.pallas_call(
        body,
        grid=(n // R,),
        in_specs=[
            pl.BlockSpec((NC, R, hh), lambda i: (0, i, 0)),
            pl.BlockSpec((R, h), lambda i: (i, 0)),
            pl.BlockSpec((R, 1), lambda i: (i, 0)),
            pl.BlockSpec((1, h), lambda i: (0, 0)),
            pl.BlockSpec((h, h2), lambda i: (0, 0)),
        ],
        out_specs=pl.BlockSpec((R, h2), lambda i: (i, 0)),
        out_shape=jax.ShapeDtypeStruct((n, h2), jnp.float32),
    )(acc, hs, dinv, b.reshape(1, h), W)


def _tc_final(acc, hs, dinv, b, bcol, Wc1, bc1, Wc2, bc2):
    n, h = hs.shape
    hh = h // NC
    c1 = Wc1.shape[1]
    c2 = Wc2.shape[1]
    R = 2000
    steps = n // R

    def body(acc_ref, hs_ref, dinv_ref, b_ref, bat_ref, wc1_ref, bc1_ref,
             wc2_ref, bc2_ref, out_ref, gsum, cnt):
        i = pl.program_id(0)

        @pl.when(i == 0)
        def _():
            gsum[...] = jnp.zeros_like(gsum)
            cnt[...] = jnp.zeros_like(cnt)

        s = jnp.concatenate([acc_ref[0], acc_ref[1]], axis=1) + hs_ref[...]
        hv = jnp.maximum(s * dinv_ref[...] + b_ref[...], 0.0)
        onehot = (bat_ref[...] ==
                  lax.broadcasted_iota(jnp.int32, (R, G), 1)).astype(jnp.float32)
        gsum[...] += lax.dot_general(onehot, hv, (((0,), (0,)), ((), ())),
                                     preferred_element_type=jnp.float32)
        cnt[...] += lax.dot_general(onehot, jnp.ones((R, 1), jnp.float32),
                                    (((0,), (0,)), ((), ())),
                                    preferred_element_type=jnp.float32)

        @pl.when(i == steps - 1)
        def _():
            g = gsum[...] / jnp.maximum(cnt[...], 1.0)
            z = jnp.maximum(jnp.dot(g, wc1_ref[...],
                                    preferred_element_type=jnp.float32)
                            + bc1_ref[...], 0.0)
            out_ref[...] = jnp.dot(z, wc2_ref[...],
                                   preferred_element_type=jnp.float32) + bc2_ref[...]

    return pl.pallas_call(
        body,
        grid=(steps,),
        in_specs=[
            pl.BlockSpec((NC, R, hh), lambda i: (0, i, 0)),
            pl.BlockSpec((R, h), lambda i: (i, 0)),
            pl.BlockSpec((R, 1), lambda i: (i, 0)),
            pl.BlockSpec((1, h), lambda i: (0, 0)),
            pl.BlockSpec((R, 1), lambda i: (i, 0)),
            pl.BlockSpec((h, c1), lambda i: (0, 0)),
            pl.BlockSpec((1, c1), lambda i: (0, 0)),
            pl.BlockSpec((c1, c2), lambda i: (0, 0)),
            pl.BlockSpec((1, c2), lambda i: (0, 0)),
        ],
        out_specs=pl.BlockSpec((G, c2), lambda i: (0, 0)),
        out_shape=jax.ShapeDtypeStruct((G, c2), jnp.float32),
        scratch_shapes=[
            pltpu.VMEM((G, h), jnp.float32),
            pltpu.VMEM((G, 1), jnp.float32),
        ],
    )(acc, hs, dinv, b.reshape(1, h), bcol, Wc1, bc1.reshape(1, c1),
      Wc2, bc2.reshape(1, c2))


def kernel(x, edge_index, batch, W1, b1, W2, b2, Wc1, bc1, Wc2, bc2):
    n, d = x.shape
    h = W1.shape[1]
    e = edge_index.shape[1]
    src, dst = edge_index[0], edge_index[1]

    KL = 4 * LANES
    EPT = _round_up(-(-e // NS), 2 * KL)
    CB = EPT // KL
    e_pad = NS * EPT
    fill = jnp.full((e_pad - e,), n, jnp.int32)
    src_p = jnp.concatenate([src, fill])
    dst_p = jnp.concatenate([dst, fill])
    src3 = src_p.reshape(NS, CB, KL)
    dst3 = dst_p.reshape(NS, CB, KL)
    src3 = jnp.concatenate(
        [src3, jnp.full((NS, 1, KL), n, jnp.int32)], axis=1)
    dst_deg = dst_p.reshape(NW, e_pad // (NW * LANES), LANES)

    np_ = _round_up(n, NS * 8)
    hh = h // NC
    ones_blk = jnp.ones((LANES, 8), jnp.float32)
    zeros_n8 = jnp.zeros((np_, 8), jnp.float32)
    zeros_nhh = jnp.zeros((np_, hh), jnp.float32)
    zrows = jnp.zeros((np_ - n, h), jnp.float32)

    degp = _sc_degree(dst_deg, ones_blk, zeros_n8, n)
    hs1, dinv = _tc_matmul_scale(x, W1, degp)
    acc1 = _sc_scatter_rows(_split_cols(jnp.concatenate([hs1, zrows]), np_),
                            src3, dst3, zeros_nhh, n, h)
    hs2 = _tc_layer(acc1, hs1, dinv, b1, W2)
    acc2 = _sc_scatter_rows(_split_cols(jnp.concatenate([hs2, zrows]), np_),
                            src3, dst3, zeros_nhh, n, h)
    return _tc_final(acc2, hs2, dinv, b2, batch.reshape(n, 1),
                     Wc1, bc1, Wc2, bc2)


# speedup vs baseline: 1.0390x; 1.0390x over previous
import functools

import jax
import jax.numpy as jnp
from jax import lax
from jax.experimental import pallas as pl
from jax.experimental.pallas import tpu as pltpu
from jax.experimental.pallas import tpu_sc as plsc

NC = 2
NS = 16
NW = NC * NS
LANES = 128
G = 64
R = 2048


def _round_up(v, m):
    return -(-v // m) * m


def _sc_degree(dst3, ones_blk, zeros_n8, n_pad):
    CH = dst3.shape[1]
    ZR = n_pad // NS
    mesh = plsc.VectorSubcoreMesh(
        core_axis_name="c", subcore_axis_name="s", num_cores=NC, num_subcores=NS)

    @functools.partial(
        pl.kernel,
        out_type=jax.ShapeDtypeStruct((NC, n_pad, 8), jnp.float32),
        mesh=mesh,
        compiler_params=pltpu.CompilerParams(use_tc_tiling_on_sc=False),
        scratch_types=[
            pltpu.VMEM((CH, LANES), jnp.int32),
            pltpu.VMEM((LANES, 8), jnp.float32),
            pltpu.VMEM_SHARED((n_pad, 8), jnp.float32),
        ],
    )
    def deg_kernel(dst_hbm, ones_hbm, zeros_hbm, out_hbm, dst_v, ones_v, deg_sh):
        cid = lax.axis_index("c")
        sid = lax.axis_index("s")
        wid = sid * NC + cid
        pltpu.sync_copy(zeros_hbm.at[pl.ds(sid * ZR, ZR)],
                        deg_sh.at[pl.ds(sid * ZR, ZR)])
        pltpu.sync_copy(ones_hbm, ones_v)
        pltpu.sync_copy(dst_hbm.at[wid], dst_v)
        plsc.subcore_barrier()

        @pl.loop(0, CH)
        def _(j):
            pltpu.sync_copy(ones_v, deg_sh.at[dst_v.at[j]], add=True)

        plsc.subcore_barrier()
        pltpu.sync_copy(deg_sh.at[pl.ds(sid * ZR, ZR)],
                        out_hbm.at[cid, pl.ds(sid * ZR, ZR)])

    return deg_kernel(dst3, ones_blk, zeros_n8)


def _sc_scatter_rows(hs_split, src3, dst3, zeros_nhh, n_pad, hh):
    CB, KL = dst3.shape[1], dst3.shape[2]
    ZR = n_pad // NS
    mesh = plsc.VectorSubcoreMesh(
        core_axis_name="c", subcore_axis_name="s", num_cores=NC, num_subcores=NS)

    @functools.partial(
        pl.kernel,
        out_type=jax.ShapeDtypeStruct((NC, n_pad, hh), jnp.float32),
        mesh=mesh,
        compiler_params=pltpu.CompilerParams(use_tc_tiling_on_sc=False),
        scratch_types=[
            pltpu.VMEM((CB + 1, KL), jnp.int32),
            pltpu.VMEM((CB, KL), jnp.int32),
            [pltpu.VMEM((KL, hh), jnp.float32) for _ in range(2)],
            pltpu.VMEM_SHARED((n_pad, hh), jnp.float32),
            pltpu.VMEM_SHARED((n_pad, hh), jnp.float32),
            [pltpu.SemaphoreType.DMA for _ in range(2)],
            [pltpu.SemaphoreType.DMA for _ in range(2)],
        ],
    )
    def msg_kernel(hs_hbm, src_hbm, dst_hbm, zeros_hbm, out_hbm,
                   src_v, dst_v, rows, hs_sh, acc_sh, semg, sems):
        cid = lax.axis_index("c")
        sid = lax.axis_index("s")
        pltpu.sync_copy(hs_hbm.at[cid, pl.ds(sid * ZR, ZR)],
                        hs_sh.at[pl.ds(sid * ZR, ZR)])
        pltpu.sync_copy(zeros_hbm.at[pl.ds(sid * ZR, ZR)],
                        acc_sh.at[pl.ds(sid * ZR, ZR)])
        pltpu.sync_copy(src_hbm.at[sid], src_v)
        pltpu.sync_copy(dst_hbm.at[sid], dst_v)
        plsc.subcore_barrier()

        def gath(c, b):
            pltpu.async_copy(hs_sh.at[src_v.at[c]], rows[b], semg[b])

        def gath_wait(c, b):
            pltpu.make_async_copy(hs_sh.at[src_v.at[c]], rows[b],
                                  semg[b]).wait()

        def scat(c, b):
            pltpu.async_copy(rows[b], acc_sh.at[dst_v.at[c]], sems[b],
                             add=True)

        def scat_wait(c, b):
            pltpu.make_async_copy(rows[b], acc_sh.at[dst_v.at[c]],
                                  sems[b]).wait()

        gath(0, 0)

        @pl.loop(0, CB, step=2)
        def _(j):
            @pl.when(j > 0)
            def _():
                scat_wait(j - 1, 1)

            gath(j + 1, 1)
            gath_wait(j, 0)
            scat(j, 0)
            scat_wait(j, 0)
            gath(j + 2, 0)
            gath_wait(j + 1, 1)
            scat(j + 1, 1)

        gath_wait(CB, 0)
        scat_wait(CB - 1, 1)
        plsc.subcore_barrier()
        pltpu.sync_copy(acc_sh.at[pl.ds(sid * ZR, ZR)],
                        out_hbm.at[cid, pl.ds(sid * ZR, ZR)])

    return msg_kernel(hs_split, src3, dst3, zeros_nhh)


def _tc_matmul_scale(xp, W, degp):
    n_pad, d = xp.shape
    h = W.shape[1]
    hh = h // NC

    def body(x_ref, w_ref, deg_ref, hss_ref, dinv_ref):
        deg = deg_ref[0, :, 0:1] + deg_ref[1, :, 0:1] + 1.0
        dinv = lax.rsqrt(deg)
        hm = jnp.dot(x_ref[...], w_ref[...], preferred_element_type=jnp.float32)
        hs = hm * dinv
        hss_ref[0] = hs[:, :hh]
        hss_ref[1] = hs[:, hh:]
        dinv_ref[...] = dinv

    return pl.pallas_call(
        body,
        grid=(n_pad // R,),
        in_specs=[
            pl.BlockSpec((R, d), lambda i: (i, 0)),
            pl.BlockSpec((d, h), lambda i: (0, 0)),
            pl.BlockSpec((NC, R, 8), lambda i: (0, i, 0)),
        ],
        out_specs=[
            pl.BlockSpec((NC, R, hh), lambda i: (0, i, 0)),
            pl.BlockSpec((R, 1), lambda i: (i, 0)),
        ],
        out_shape=[
            jax.ShapeDtypeStruct((NC, n_pad, hh), jnp.float32),
            jax.ShapeDtypeStruct((n_pad, 1), jnp.float32),
        ],
    )(xp, W, degp)


def _tc_layer(acc, hss, dinv, b, W):
    _, n_pad, hh = hss.shape
    h = hh * NC
    h2 = W.shape[1]
    hh2 = h2 // NC

    def body(acc_ref, hss_ref, dinv_ref, b_ref, w_ref, out_ref):
        s = jnp.concatenate(
            [acc_ref[0] + hss_ref[0], acc_ref[1] + hss_ref[1]], axis=1)
        h1 = jnp.maximum(s * dinv_ref[...] + b_ref[...], 0.0)
        hm = jnp.dot(h1, w_ref[...],
                     preferred_element_type=jnp.float32) * dinv_ref[...]
        out_ref[0] = hm[:, :hh2]
        out_ref[1] = hm[:, hh2:]

    return pl.pallas_call(
        body,
        grid=(n_pad // R,),
        in_specs=[
            pl.BlockSpec((NC, R, hh), lambda i: (0, i, 0)),
            pl.BlockSpec((NC, R, hh), lambda i: (0, i, 0)),
            pl.BlockSpec((R, 1), lambda i: (i, 0)),
            pl.BlockSpec((1, h), lambda i: (0, 0)),
            pl.BlockSpec((h, h2), lambda i: (0, 0)),
        ],
        out_specs=pl.BlockSpec((NC, R, hh2), lambda i: (0, i, 0)),
        out_shape=jax.ShapeDtypeStruct((NC, n_pad, hh2), jnp.float32),
    )(acc, hss, dinv, b.reshape(1, h), W)


def _tc_final(acc, hss, dinv, b, bcol, Wc1, bc1, Wc2, bc2):
    _, n_pad, hh = hss.shape
    h = hh * NC
    c1 = Wc1.shape[1]
    c2 = Wc2.shape[1]
    steps = n_pad // R

    def body(acc_ref, hss_ref, dinv_ref, b_ref, bat_ref, wc1_ref, bc1_ref,
             wc2_ref, bc2_ref, out_ref, gsum, cnt):
        i = pl.program_id(0)

        @pl.when(i == 0)
        def _():
            gsum[...] = jnp.zeros_like(gsum)
            cnt[...] = jnp.zeros_like(cnt)

        s = jnp.concatenate(
            [acc_ref[0] + hss_ref[0], acc_ref[1] + hss_ref[1]], axis=1)
        hv = jnp.maximum(s * dinv_ref[...] + b_ref[...], 0.0)
        onehot = (bat_ref[...] ==
                  lax.broadcasted_iota(jnp.int32, (R, G), 1)).astype(jnp.float32)
        gsum[...] += lax.dot_general(onehot, hv, (((0,), (0,)), ((), ())),
                                     preferred_element_type=jnp.float32)
        cnt[...] += lax.dot_general(onehot, jnp.ones((R, 1), jnp.float32),
                                    (((0,), (0,)), ((), ())),
                                    preferred_element_type=jnp.float32)

        @pl.when(i == steps - 1)
        def _():
            g = gsum[...] / jnp.maximum(cnt[...], 1.0)
            z = jnp.maximum(jnp.dot(g, wc1_ref[...],
                                    preferred_element_type=jnp.float32)
                            + bc1_ref[...], 0.0)
            out_ref[...] = jnp.dot(z, wc2_ref[...],
                                   preferred_element_type=jnp.float32) + bc2_ref[...]

    return pl.pallas_call(
        body,
        grid=(steps,),
        in_specs=[
            pl.BlockSpec((NC, R, hh), lambda i: (0, i, 0)),
            pl.BlockSpec((NC, R, hh), lambda i: (0, i, 0)),
            pl.BlockSpec((R, 1), lambda i: (i, 0)),
            pl.BlockSpec((1, h), lambda i: (0, 0)),
            pl.BlockSpec((R, 1), lambda i: (i, 0)),
            pl.BlockSpec((h, c1), lambda i: (0, 0)),
            pl.BlockSpec((1, c1), lambda i: (0, 0)),
            pl.BlockSpec((c1, c2), lambda i: (0, 0)),
            pl.BlockSpec((1, c2), lambda i: (0, 0)),
        ],
        out_specs=pl.BlockSpec((G, c2), lambda i: (0, 0)),
        out_shape=jax.ShapeDtypeStruct((G, c2), jnp.float32),
        scratch_shapes=[
            pltpu.VMEM((G, h), jnp.float32),
            pltpu.VMEM((G, 1), jnp.float32),
        ],
    )(acc, hss, dinv, b.reshape(1, h), bcol, Wc1, bc1.reshape(1, c1),
      Wc2, bc2.reshape(1, c2))


def kernel(x, edge_index, batch, W1, b1, W2, b2, Wc1, bc1, Wc2, bc2):
    n, d = x.shape
    h = W1.shape[1]
    hh = h // NC
    e = edge_index.shape[1]
    src, dst = edge_index[0], edge_index[1]

    n_pad = _round_up(n, R)
    KL = 4 * LANES
    EPT = _round_up(-(-e // NS), 2 * KL)
    CB = EPT // KL
    e_pad = NS * EPT
    fill = jnp.full((e_pad - e,), n, jnp.int32)
    src_p = jnp.concatenate([src, fill])
    dst_p = jnp.concatenate([dst, fill])
    src3 = src_p.reshape(NS, CB, KL)
    dst3 = dst_p.reshape(NS, CB, KL)
    src3 = jnp.concatenate(
        [src3, jnp.full((NS, 1, KL), n, jnp.int32)], axis=1)
    dst_deg = dst_p.reshape(NW, e_pad // (NW * LANES), LANES)

    xp = jnp.concatenate([x, jnp.zeros((n_pad - n, d), jnp.float32)])
    bcol = jnp.concatenate([batch, jnp.full((n_pad - n,), G, jnp.int32)])
    ones_blk = jnp.ones((LANES, 8), jnp.float32)
    zeros_n8 = jnp.zeros((n_pad, 8), jnp.float32)
    zeros_nhh = jnp.zeros((n_pad, hh), jnp.float32)

    degp = _sc_degree(dst_deg, ones_blk, zeros_n8, n_pad)
    hs1s, dinv = _tc_matmul_scale(xp, W1, degp)
    acc1 = _sc_scatter_rows(hs1s, src3, dst3, zeros_nhh, n_pad, hh)
    hs2s = _tc_layer(acc1, hs1s, dinv, b1, W2)
    acc2 = _sc_scatter_rows(hs2s, src3, dst3, zeros_nhh, n_pad, hh)
    return _tc_final(acc2, hs2s, dinv, b2, bcol.reshape(n_pad, 1),
                     Wc1, bc1, Wc2, bc2)


# speedup vs baseline: 1.0399x; 1.0009x over previous
import functools

import jax
import jax.numpy as jnp
from jax import lax
from jax.experimental import pallas as pl
from jax.experimental.pallas import tpu as pltpu
from jax.experimental.pallas import tpu_sc as plsc

NC = 2
NS = 16
NW = NC * NS
LANES = 128
G = 64
R = 2048


def _round_up(v, m):
    return -(-v // m) * m


def _sc_degree(dst3, ones_blk, zeros_n8, n_pad):
    CH, KLd = dst3.shape[1], dst3.shape[2]
    ZR = n_pad // NS
    mesh = plsc.VectorSubcoreMesh(
        core_axis_name="c", subcore_axis_name="s", num_cores=NC, num_subcores=NS)

    @functools.partial(
        pl.kernel,
        out_type=jax.ShapeDtypeStruct((NC, n_pad, 8), jnp.float32),
        mesh=mesh,
        compiler_params=pltpu.CompilerParams(use_tc_tiling_on_sc=False),
        scratch_types=[
            pltpu.VMEM((CH, KLd), jnp.int32),
            pltpu.VMEM((KLd, 8), jnp.float32),
            pltpu.VMEM_SHARED((n_pad, 8), jnp.float32),
        ],
    )
    def deg_kernel(dst_hbm, ones_hbm, zeros_hbm, out_hbm, dst_v, ones_v, deg_sh):
        cid = lax.axis_index("c")
        sid = lax.axis_index("s")
        wid = sid * NC + cid
        pltpu.sync_copy(zeros_hbm.at[pl.ds(sid * ZR, ZR)],
                        deg_sh.at[pl.ds(sid * ZR, ZR)])
        pltpu.sync_copy(ones_hbm, ones_v)
        pltpu.sync_copy(dst_hbm.at[wid], dst_v)
        plsc.subcore_barrier()

        @pl.loop(0, CH)
        def _(j):
            pltpu.sync_copy(ones_v, deg_sh.at[dst_v.at[j]], add=True)

        plsc.subcore_barrier()
        pltpu.sync_copy(deg_sh.at[pl.ds(sid * ZR, ZR)],
                        out_hbm.at[cid, pl.ds(sid * ZR, ZR)])

    return deg_kernel(dst3, ones_blk, zeros_n8)


def _sc_scatter_rows(hs_split, src3, dst3, zeros_nhh, n_pad, hh):
    CB, KL = dst3.shape[1], dst3.shape[2]
    ZR = n_pad // NS
    mesh = plsc.VectorSubcoreMesh(
        core_axis_name="c", subcore_axis_name="s", num_cores=NC, num_subcores=NS)

    @functools.partial(
        pl.kernel,
        out_type=jax.ShapeDtypeStruct((NC, n_pad, hh), jnp.float32),
        mesh=mesh,
        compiler_params=pltpu.CompilerParams(use_tc_tiling_on_sc=False),
        scratch_types=[
            pltpu.VMEM((CB + 1, KL), jnp.int32),
            pltpu.VMEM((CB, KL), jnp.int32),
            [pltpu.VMEM((KL, hh), jnp.float32) for _ in range(2)],
            pltpu.VMEM_SHARED((n_pad, hh), jnp.float32),
            pltpu.VMEM_SHARED((n_pad, hh), jnp.float32),
            [pltpu.SemaphoreType.DMA for _ in range(2)],
            [pltpu.SemaphoreType.DMA for _ in range(2)],
        ],
    )
    def msg_kernel(hs_hbm, src_hbm, dst_hbm, zeros_hbm, out_hbm,
                   src_v, dst_v, rows, hs_sh, acc_sh, semg, sems):
        cid = lax.axis_index("c")
        sid = lax.axis_index("s")
        pltpu.sync_copy(hs_hbm.at[cid, pl.ds(sid * ZR, ZR)],
                        hs_sh.at[pl.ds(sid * ZR, ZR)])
        pltpu.sync_copy(zeros_hbm.at[pl.ds(sid * ZR, ZR)],
                        acc_sh.at[pl.ds(sid * ZR, ZR)])
        pltpu.sync_copy(src_hbm.at[sid], src_v)
        pltpu.sync_copy(dst_hbm.at[sid], dst_v)
        plsc.subcore_barrier()

        def gath(c, b):
            pltpu.async_copy(hs_sh.at[src_v.at[c]], rows[b], semg[b])

        def gath_wait(c, b):
            pltpu.make_async_copy(hs_sh.at[src_v.at[c]], rows[b],
                                  semg[b]).wait()

        def scat(c, b):
            pltpu.async_copy(rows[b], acc_sh.at[dst_v.at[c]], sems[b],
                             add=True)

        def scat_wait(c, b):
            pltpu.make_async_copy(rows[b], acc_sh.at[dst_v.at[c]],
                                  sems[b]).wait()

        gath(0, 0)

        @pl.loop(0, CB, step=2)
        def _(j):
            @pl.when(j > 0)
            def _():
                scat_wait(j - 1, 1)

            gath(j + 1, 1)
            gath_wait(j, 0)
            scat(j, 0)
            scat_wait(j, 0)
            gath(j + 2, 0)
            gath_wait(j + 1, 1)
            scat(j + 1, 1)

        gath_wait(CB, 0)
        scat_wait(CB - 1, 1)
        plsc.subcore_barrier()
        pltpu.sync_copy(acc_sh.at[pl.ds(sid * ZR, ZR)],
                        out_hbm.at[cid, pl.ds(sid * ZR, ZR)])

    return msg_kernel(hs_split, src3, dst3, zeros_nhh)


def _tc_matmul(xp, W):
    n_pad, d = xp.shape
    h = W.shape[1]

    def body(x_ref, w_ref, out_ref):
        out_ref[...] = jnp.dot(x_ref[...], w_ref[...],
                               preferred_element_type=jnp.float32)

    return pl.pallas_call(
        body,
        grid=(n_pad // R,),
        in_specs=[
            pl.BlockSpec((R, d), lambda i: (i, 0)),
            pl.BlockSpec((d, h), lambda i: (0, 0)),
        ],
        out_specs=pl.BlockSpec((R, h), lambda i: (i, 0)),
        out_shape=jax.ShapeDtypeStruct((n_pad, h), jnp.float32),
    )(xp, W)


def _tc_scale(mm, degp):
    n_pad, h = mm.shape
    hh = h // NC

    def body(mm_ref, deg_ref, hss_ref, dinv_ref):
        deg = deg_ref[0, :, 0:1] + deg_ref[1, :, 0:1] + 1.0
        dinv = lax.rsqrt(deg)
        hs = mm_ref[...] * dinv
        hss_ref[0] = hs[:, :hh]
        hss_ref[1] = hs[:, hh:]
        dinv_ref[...] = dinv

    return pl.pallas_call(
        body,
        grid=(n_pad // R,),
        in_specs=[
            pl.BlockSpec((R, h), lambda i: (i, 0)),
            pl.BlockSpec((NC, R, 8), lambda i: (0, i, 0)),
        ],
        out_specs=[
            pl.BlockSpec((NC, R, hh), lambda i: (0, i, 0)),
            pl.BlockSpec((R, 1), lambda i: (i, 0)),
        ],
        out_shape=[
            jax.ShapeDtypeStruct((NC, n_pad, hh), jnp.float32),
            jax.ShapeDtypeStruct((n_pad, 1), jnp.float32),
        ],
    )(mm, degp)


def _tc_layer(acc, hss, dinv, b, W):
    _, n_pad, hh = hss.shape
    h = hh * NC
    h2 = W.shape[1]
    hh2 = h2 // NC

    def body(acc_ref, hss_ref, dinv_ref, b_ref, w_ref, out_ref):
        s = jnp.concatenate(
            [acc_ref[0] + hss_ref[0], acc_ref[1] + hss_ref[1]], axis=1)
        h1 = jnp.maximum(s * dinv_ref[...] + b_ref[...], 0.0)
        hm = jnp.dot(h1, w_ref[...],
                     preferred_element_type=jnp.float32) * dinv_ref[...]
        out_ref[0] = hm[:, :hh2]
        out_ref[1] = hm[:, hh2:]

    return pl.pallas_call(
        body,
        grid=(n_pad // R,),
        in_specs=[
            pl.BlockSpec((NC, R, hh), lambda i: (0, i, 0)),
            pl.BlockSpec((NC, R, hh), lambda i: (0, i, 0)),
            pl.BlockSpec((R, 1), lambda i: (i, 0)),
            pl.BlockSpec((1, h), lambda i: (0, 0)),
            pl.BlockSpec((h, h2), lambda i: (0, 0)),
        ],
        out_specs=pl.BlockSpec((NC, R, hh2), lambda i: (0, i, 0)),
        out_shape=jax.ShapeDtypeStruct((NC, n_pad, hh2), jnp.float32),
    )(acc, hss, dinv, b.reshape(1, h), W)


def _tc_final(acc, hss, dinv, b, bcol, Wc1, bc1, Wc2, bc2):
    _, n_pad, hh = hss.shape
    h = hh * NC
    c1 = Wc1.shape[1]
    c2 = Wc2.shape[1]
    steps = n_pad // R

    def body(acc_ref, hss_ref, dinv_ref, b_ref, bat_ref, wc1_ref, bc1_ref,
             wc2_ref, bc2_ref, out_ref, gsum, cnt):
        i = pl.program_id(0)

        @pl.when(i == 0)
        def _():
            gsum[...] = jnp.zeros_like(gsum)
            cnt[...] = jnp.zeros_like(cnt)

        s = jnp.concatenate(
            [acc_ref[0] + hss_ref[0], acc_ref[1] + hss_ref[1]], axis=1)
        hv = jnp.maximum(s * dinv_ref[...] + b_ref[...], 0.0)
        onehot = (bat_ref[...] ==
                  lax.broadcasted_iota(jnp.int32, (R, G), 1)).astype(jnp.float32)
        gsum[...] += lax.dot_general(onehot, hv, (((0,), (0,)), ((), ())),
                                     preferred_element_type=jnp.float32)
        cnt[...] += lax.dot_general(onehot, jnp.ones((R, 1), jnp.float32),
                                    (((0,), (0,)), ((), ())),
                                    preferred_element_type=jnp.float32)

        @pl.when(i == steps - 1)
        def _():
            g = gsum[...] / jnp.maximum(cnt[...], 1.0)
            z = jnp.maximum(jnp.dot(g, wc1_ref[...],
                                    preferred_element_type=jnp.float32)
                            + bc1_ref[...], 0.0)
            out_ref[...] = jnp.dot(z, wc2_ref[...],
                                   preferred_element_type=jnp.float32) + bc2_ref[...]

    return pl.pallas_call(
        body,
        grid=(steps,),
        in_specs=[
            pl.BlockSpec((NC, R, hh), lambda i: (0, i, 0)),
            pl.BlockSpec((NC, R, hh), lambda i: (0, i, 0)),
            pl.BlockSpec((R, 1), lambda i: (i, 0)),
            pl.BlockSpec((1, h), lambda i: (0, 0)),
            pl.BlockSpec((R, 1), lambda i: (i, 0)),
            pl.BlockSpec((h, c1), lambda i: (0, 0)),
            pl.BlockSpec((1, c1), lambda i: (0, 0)),
            pl.BlockSpec((c1, c2), lambda i: (0, 0)),
            pl.BlockSpec((1, c2), lambda i: (0, 0)),
        ],
        out_specs=pl.BlockSpec((G, c2), lambda i: (0, 0)),
        out_shape=jax.ShapeDtypeStruct((G, c2), jnp.float32),
        scratch_shapes=[
            pltpu.VMEM((G, h), jnp.float32),
            pltpu.VMEM((G, 1), jnp.float32),
        ],
    )(acc, hss, dinv, b.reshape(1, h), bcol, Wc1, bc1.reshape(1, c1),
      Wc2, bc2.reshape(1, c2))


def kernel(x, edge_index, batch, W1, b1, W2, b2, Wc1, bc1, Wc2, bc2):
    n, d = x.shape
    h = W1.shape[1]
    hh = h // NC
    e = edge_index.shape[1]
    src, dst = edge_index[0], edge_index[1]

    n_pad = _round_up(n, R)
    KL = 4 * LANES
    EPT = _round_up(-(-e // NS), 2 * KL)
    CB = EPT // KL
    e_pad = NS * EPT
    fill = jnp.full((e_pad - e,), n, jnp.int32)
    src_p = jnp.concatenate([src, fill])
    dst_p = jnp.concatenate([dst, fill])
    src3 = src_p.reshape(NS, CB, KL)
    dst3 = dst_p.reshape(NS, CB, KL)
    src3 = jnp.concatenate(
        [src3, jnp.full((NS, 1, KL), n, jnp.int32)], axis=1)
    KLd = 4 * LANES
    dst_deg = dst_p.reshape(NW, e_pad // (NW * KLd), KLd)

    xp = jnp.concatenate([x, jnp.zeros((n_pad - n, d), jnp.float32)])
    bcol = jnp.concatenate([batch, jnp.full((n_pad - n,), G, jnp.int32)])
    ones_blk = jnp.ones((KLd, 8), jnp.float32)
    zeros_n8 = jnp.zeros((n_pad, 8), jnp.float32)
    zeros_nhh = jnp.zeros((n_pad, hh), jnp.float32)

    mm1 = _tc_matmul(xp, W1)
    degp = _sc_degree(dst_deg, ones_blk, zeros_n8, n_pad)
    hs1s, dinv = _tc_scale(mm1, degp)
    acc1 = _sc_scatter_rows(hs1s, src3, dst3, zeros_nhh, n_pad, hh)
    hs2s = _tc_layer(acc1, hs1s, dinv, b1, W2)
    acc2 = _sc_scatter_rows(hs2s, src3, dst3, zeros_nhh, n_pad, hh)
    return _tc_final(acc2, hs2s, dinv, b2, bcol.reshape(n_pad, 1),
                     Wc1, bc1, Wc2, bc2)
